# baseline (speedup 1.0000x reference)
import math

import jax
import jax.numpy as jnp
from jax import lax
from jax.experimental import pallas as pl
from jax.experimental.pallas import tpu as pltpu

B = 8
N = 197
D = 192
NH = 3
DH = 64
L = 6
NE = 16
HD = 768
CAP = 197
CAPP = 256
NP = 256
BNP = B * NP
EC = NE * CAPP

_f32 = jnp.float32
_bf16 = jnp.bfloat16


def _bdot(a, b):
    return lax.dot_general(
        a.astype(_bf16), b.astype(_bf16), (((a.ndim - 1,), (0,)), ((), ())),
        preferred_element_type=_f32)


def _ln(x, g, b):
    m = x.mean(-1, keepdims=True)
    v = ((x - m) ** 2).mean(-1, keepdims=True)
    return (x - m) * lax.rsqrt(v + 1e-6) * g + b


def _softmax(s):
    p = jnp.exp(s)
    return p * (1.0 / jnp.sum(p, axis=-1, keepdims=True))


def _embed_body(xp_ref, wp_ref, bp_ref, cls_ref, pos_ref, out_ref):
    y = jnp.dot(xp_ref[...], wp_ref[...]) + bp_ref[...]
    for b in range(B):
        out_ref[b, 0:1, :] = cls_ref[...] + pos_ref[0:1, :]
        out_ref[b, 1:N, :] = y[b * (N - 1):(b + 1) * (N - 1), :] + pos_ref[1:N, :]


def _embed(xp, wp, bp, cls, pos):
    return pl.pallas_call(
        _embed_body,
        out_shape=jax.ShapeDtypeStruct((B, N, D), _f32),
    )(xp, wp, bp, cls, pos)


def _encoder_body(t0_ref, ln1g_ref, ln1b_ref, wqkv_ref, bqkv_ref, wproj_ref,
                  bproj_ref, ln2g_ref, ln2b_ref, wg_ref, w1_ref, b1_ref,
                  w2_ref, b2_ref, lnfg_ref, lnfb_ref, wh_ref, bh_ref,
                  logits_ref, cv_ref,
                  t_s, u_s, a_s, o_s, scl_s, aux_s):
    l = pl.program_id(0)
    e = pl.program_id(1)

    @pl.when(jnp.logical_and(l == 0, e == 0))
    def _zero_pad():
        for b in range(B):
            u_s[pl.ds(b * NP + N, NP - N), :] = jnp.zeros((NP - N, D), _bf16)
            a_s[pl.ds(b * NP + N, NP - N), :] = jnp.zeros((NP - N, EC), _bf16)

    @pl.when(e == 0)
    def _prologue():
        iota_e = lax.broadcasted_iota(jnp.int32, (N, NE), 1).astype(_f32)
        ii = lax.broadcasted_iota(jnp.int32, (N, N), 0)
        jj = lax.broadcasted_iota(jnp.int32, (N, N), 1)
        tril = (jj < ii).astype(_bf16)
        iota_ec = lax.broadcasted_iota(jnp.int32, (N, EC), 1)
        is_l0 = l == 0

        off = jnp.zeros((NE,), _f32)
        imp = jnp.zeros((NE,), _f32)
        for b in range(B):
            sb = scl_s[pl.ds(b * NP, N), :]
            yrows = jnp.dot(a_s[pl.ds(b * NP, N), :], o_s[...],
                            preferred_element_type=_f32)
            fold = t_s[b, :N, :] + jnp.where(sb > 0.0, yrows * sb, 0.0)
            tb = jnp.where(is_l0, t0_ref[b], fold)
            u1 = _ln(tb, ln1g_ref[0], ln1b_ref[0])
            qkv = _bdot(u1, wqkv_ref[0]) + bqkv_ref[0]
            outs = []
            for h in range(NH):
                q = qkv[:, h * DH:(h + 1) * DH]
                k = qkv[:, D + h * DH:D + (h + 1) * DH]
                v = qkv[:, 2 * D + h * DH:2 * D + (h + 1) * DH]
                s = lax.dot_general(
                    q.astype(_bf16), k.astype(_bf16),
                    (((1,), (1,)), ((), ())),
                    preferred_element_type=_f32) * (1.0 / math.sqrt(DH))
                p = _softmax(s)
                outs.append(_bdot(p, v))
            o = jnp.concatenate(outs, axis=1)
            tm = tb + _bdot(o, wproj_ref[0]) + bproj_ref[0]
            t_s[b, :N, :] = tm
            u2 = _ln(tm, ln2g_ref[0], ln2b_ref[0])
            u_s[pl.ds(b * NP, N), :] = u2.astype(_bf16)

            logits = jnp.dot(u2, wg_ref[0])
            probs = _softmax(logits)
            gate = jnp.max(probs, axis=-1)
            eq = probs == gate[:, None]
            idxf = jnp.min(jnp.where(eq, iota_e, 1e9), axis=-1)
            oh = (iota_e == idxf[:, None]).astype(_f32)
            cnt = _bdot(tril, oh)
            pos = jnp.sum((cnt + off[None, :]) * oh, axis=-1)
            keep = (pos < float(CAP)).astype(_f32)
            posc = jnp.minimum(pos, float(CAPP - 1))
            loc = (idxf * float(CAPP) + posc).astype(jnp.int32)
            scl_s[pl.ds(b * NP, N), :] = (gate * keep)[:, None]
            a_s[pl.ds(b * NP, N), :] = (loc[:, None] == iota_ec).astype(_bf16)
            off = off + jnp.sum(oh, axis=0)
            imp = imp + jnp.sum(probs, axis=0)

        tot = float(B * N)
        aux = float(NE) * jnp.sum((imp / tot) * (off / tot))
        prev = aux_s[0, 0]
        aux_s[...] = jnp.where(is_l0, aux, prev + aux).reshape(1, 1)

    acol = a_s[:, pl.ds(pl.multiple_of(e * CAPP, CAPP), CAPP)]
    x = lax.dot_general(acol, u_s[...], (((0,), (0,)), ((), ())),
                        preferred_element_type=_f32)
    h = jax.nn.gelu(_bdot(x, w1_ref[0, 0]) + b1_ref[0, 0])
    o = _bdot(h, w2_ref[0, 0]) + b2_ref[0, 0]
    o_s[pl.ds(pl.multiple_of(e * CAPP, CAPP), CAPP), :] = o.astype(_bf16)

    @pl.when(jnp.logical_and(l == L - 1, e == NE - 1))
    def _head():
        rows = []
        for b in range(B):
            sb = scl_s[pl.ds(b * NP, 1), :]
            ycls = jnp.dot(a_s[pl.ds(b * NP, 1), :], o_s[...],
                           preferred_element_type=_f32)
            tb = t_s[b, 0:1, :] + jnp.where(sb > 0.0, ycls * sb, 0.0)
            rows.append(tb)
        tc = _ln(jnp.concatenate(rows, axis=0), lnfg_ref[0], lnfb_ref[0])
        logits_ref[...] = jnp.dot(tc, wh_ref[...]) + bh_ref[...]
        cv_ref[...] = aux_s[...]


def _encoder(t0, p):
    specs = [
        pl.BlockSpec((B, N, D), lambda l, e: (0, 0, 0)),
        pl.BlockSpec((1, 1, D), lambda l, e: (l, 0, 0)),
        pl.BlockSpec((1, 1, D), lambda l, e: (l, 0, 0)),
        pl.BlockSpec((1, D, 3 * D), lambda l, e: (l, 0, 0)),
        pl.BlockSpec((1, 1, 3 * D), lambda l, e: (l, 0, 0)),
        pl.BlockSpec((1, D, D), lambda l, e: (l, 0, 0)),
        pl.BlockSpec((1, 1, D), lambda l, e: (l, 0, 0)),
        pl.BlockSpec((1, 1, D), lambda l, e: (l, 0, 0)),
        pl.BlockSpec((1, 1, D), lambda l, e: (l, 0, 0)),
        pl.BlockSpec((1, D, NE), lambda l, e: (l, 0, 0)),
        pl.BlockSpec((1, 1, D, HD), lambda l, e: (l, e, 0, 0)),
        pl.BlockSpec((1, 1, 1, HD), lambda l, e: (l, e, 0, 0)),
        pl.BlockSpec((1, 1, HD, D), lambda l, e: (l, e, 0, 0)),
        pl.BlockSpec((1, 1, 1, D), lambda l, e: (l, e, 0, 0)),
        pl.BlockSpec((1, D), lambda l, e: (0, 0)),
        pl.BlockSpec((1, D), lambda l, e: (0, 0)),
        pl.BlockSpec((D, 1000), lambda l, e: (0, 0)),
        pl.BlockSpec((1, 1000), lambda l, e: (0, 0)),
    ]
    return pl.pallas_call(
        _encoder_body,
        grid=(L, NE),
        in_specs=specs,
        out_specs=(
            pl.BlockSpec((B, 1000), lambda l, e: (0, 0)),
            pl.BlockSpec((1, 1), lambda l, e: (0, 0)),
        ),
        out_shape=(
            jax.ShapeDtypeStruct((B, 1000), _f32),
            jax.ShapeDtypeStruct((1, 1), _f32),
        ),
        scratch_shapes=[
            pltpu.VMEM((B, NP, D), _f32),
            pltpu.VMEM((BNP, D), _bf16),
            pltpu.VMEM((BNP, EC), _bf16),
            pltpu.VMEM((EC, D), _bf16),
            pltpu.VMEM((BNP, 1), _f32),
            pltpu.VMEM((1, 1), _f32),
        ],
    )(
        t0,
        p['ln1_g'].reshape(L, 1, D), p['ln1_b'].reshape(L, 1, D),
        p['Wqkv'].astype(_bf16), p['bqkv'].reshape(L, 1, 3 * D),
        p['Wproj'].astype(_bf16), p['bproj'].reshape(L, 1, D),
        p['ln2_g'].reshape(L, 1, D), p['ln2_b'].reshape(L, 1, D),
        p['Wg'],
        p['W1'].astype(_bf16), p['b1'].reshape(L, NE, 1, HD),
        p['W2'].astype(_bf16), p['b2'].reshape(L, NE, 1, D),
        p['lnf_g'].reshape(1, D), p['lnf_b'].reshape(1, D),
        p['Whead'], p['bhead'].reshape(1, 1000),
    )


def kernel(x, params):
    p = params
    gh = 224 // 16
    xp = (x.reshape(B, 3, gh, 16, gh, 16)
           .transpose(0, 2, 4, 1, 3, 5)
           .reshape(B * gh * gh, 3 * 16 * 16))
    t0 = _embed(xp, p['Wpatch'], p['bpatch'], p['cls'].reshape(1, D),
                p['pos'].reshape(N, D))
    logits, cv = _encoder(t0, p)
    return logits, cv.reshape(())


# speedup vs baseline: 1.0324x; 1.0324x over previous
import math

import jax
import jax.numpy as jnp
from jax import lax
from jax.experimental import pallas as pl
from jax.experimental.pallas import tpu as pltpu

B = 8
N = 197
D = 192
NH = 3
DH = 64
L = 6
NE = 16
HD = 768
CAP = 197
CAPP = 208
NP = 256
BNP = B * NP

_f32 = jnp.float32
_bf16 = jnp.bfloat16


def _bdot(a, b):
    return lax.dot_general(
        a.astype(_bf16), b.astype(_bf16), (((a.ndim - 1,), (0,)), ((), ())),
        preferred_element_type=_f32)


def _ln(x, g, b):
    m = x.mean(-1, keepdims=True)
    v = ((x - m) ** 2).mean(-1, keepdims=True)
    return (x - m) * lax.rsqrt(v + 1e-6) * g + b


def _softmax(s):
    p = jnp.exp(s)
    return p * (1.0 / jnp.sum(p, axis=-1, keepdims=True))


def _embed_body(xp_ref, wp_ref, bp_ref, cls_ref, pos_ref, out_ref):
    y = jnp.dot(xp_ref[...], wp_ref[...]) + bp_ref[...]
    for b in range(B):
        out_ref[b, 0:1, :] = cls_ref[...] + pos_ref[0:1, :]
        out_ref[b, 1:N, :] = y[b * (N - 1):(b + 1) * (N - 1), :] + pos_ref[1:N, :]


def _embed(xp, wp, bp, cls, pos):
    return pl.pallas_call(
        _embed_body,
        out_shape=jax.ShapeDtypeStruct((B, N, D), _f32),
    )(xp, wp, bp, cls, pos)


def _encoder_body(t0_ref, ln1g_ref, ln1b_ref, wqkv_ref, bqkv_ref, wproj_ref,
                  bproj_ref, ln2g_ref, ln2b_ref, wg_ref, w1_ref, b1_ref,
                  w2_ref, b2_ref, lnfg_ref, lnfb_ref, wh_ref, bh_ref,
                  logits_ref, cv_ref,
                  t_s, u_s, y_s, a_s, idx_s, scl_s, aux_s):
    l = pl.program_id(0)
    e = pl.program_id(1)

    @pl.when(jnp.logical_and(l == 0, e == 0))
    def _zero_pad():
        for b in range(B):
            u_s[pl.ds(b * NP + N, NP - N), :] = jnp.zeros((NP - N, D), _bf16)
            a_s[pl.ds(b * NP + N, NP - N), :] = jnp.zeros((NP - N, CAPP), _bf16)
            idx_s[pl.ds(b * NP + N, NP - N)] = jnp.full((NP - N,), NE, jnp.int32)

    @pl.when(e == 0)
    def _prologue():
        iota_e = lax.broadcasted_iota(jnp.int32, (N, NE), 1).astype(_f32)
        ii = lax.broadcasted_iota(jnp.int32, (N, N), 0)
        jj = lax.broadcasted_iota(jnp.int32, (N, N), 1)
        tril = (jj < ii).astype(_bf16)
        iota_c = lax.broadcasted_iota(jnp.int32, (N, CAPP), 1)
        is_l0 = l == 0

        off = jnp.zeros((NE,), _f32)
        imp = jnp.zeros((NE,), _f32)
        for b in range(B):
            sb = scl_s[pl.ds(b * NP, N), :]
            yrows = y_s[pl.ds(b * NP, N), :].astype(_f32)
            fold = t_s[b, :N, :] + jnp.where(sb > 0.0, yrows * sb, 0.0)
            tb = jnp.where(is_l0, t0_ref[b], fold)
            u1 = _ln(tb, ln1g_ref[0], ln1b_ref[0])
            qkv = _bdot(u1, wqkv_ref[0]) + bqkv_ref[0]
            outs = []
            for h in range(NH):
                q = qkv[:, h * DH:(h + 1) * DH]
                k = qkv[:, D + h * DH:D + (h + 1) * DH]
                v = qkv[:, 2 * D + h * DH:2 * D + (h + 1) * DH]
                s = lax.dot_general(
                    q.astype(_bf16), k.astype(_bf16),
                    (((1,), (1,)), ((), ())),
                    preferred_element_type=_f32) * (1.0 / math.sqrt(DH))
                p = _softmax(s)
                outs.append(_bdot(p, v))
            o = jnp.concatenate(outs, axis=1)
            tm = tb + _bdot(o, wproj_ref[0]) + bproj_ref[0]
            t_s[b, :N, :] = tm
            u2 = _ln(tm, ln2g_ref[0], ln2b_ref[0])
            u_s[pl.ds(b * NP, N), :] = u2.astype(_bf16)

            logits = jnp.dot(u2, wg_ref[0])
            probs = _softmax(logits)
            gate = jnp.max(probs, axis=-1)
            eq = probs == gate[:, None]
            idxf = jnp.min(jnp.where(eq, iota_e, 1e9), axis=-1)
            oh = (iota_e == idxf[:, None]).astype(_f32)
            cnt = _bdot(tril, oh)
            pos = jnp.sum((cnt + off[None, :]) * oh, axis=-1)
            keep = (pos < float(CAP)).astype(_f32)
            posc = jnp.minimum(pos, float(CAPP - 1)).astype(jnp.int32)
            scl_s[pl.ds(b * NP, N), :] = (gate * keep)[:, None]
            a_s[pl.ds(b * NP, N), :] = (posc[:, None] == iota_c).astype(_bf16)
            idx_s[pl.ds(b * NP, N)] = idxf.astype(jnp.int32)
            off = off + jnp.sum(oh, axis=0)
            imp = imp + jnp.sum(probs, axis=0)

        tot = float(B * N)
        aux = float(NE) * jnp.sum((imp / tot) * (off / tot))
        prev = aux_s[0, 0]
        aux_s[...] = jnp.where(is_l0, aux, prev + aux).reshape(1, 1)

    mf = (idx_s[...] == e).astype(_f32)[:, None]
    m = mf.astype(_bf16)
    um = u_s[...] * m
    x = lax.dot_general(a_s[...], um, (((0,), (0,)), ((), ())),
                        preferred_element_type=_f32)
    h = jax.nn.gelu(_bdot(x, w1_ref[0, 0]) + b1_ref[0, 0])
    o = _bdot(h, w2_ref[0, 0]) + b2_ref[0, 0]
    yb = jnp.dot(a_s[...], o.astype(_bf16),
                 preferred_element_type=_f32)
    ym = (yb * mf).astype(_bf16)

    @pl.when(e == 0)
    def _():
        y_s[...] = ym

    @pl.when(e > 0)
    def _():
        y_s[...] += ym

    @pl.when(jnp.logical_and(l == L - 1, e == NE - 1))
    def _head():
        rows = []
        for b in range(B):
            sb = scl_s[pl.ds(b * NP, 1), :]
            ycls = y_s[pl.ds(b * NP, 1), :].astype(_f32)
            tb = t_s[b, 0:1, :] + jnp.where(sb > 0.0, ycls * sb, 0.0)
            rows.append(tb)
        tc = _ln(jnp.concatenate(rows, axis=0), lnfg_ref[0], lnfb_ref[0])
        logits_ref[...] = jnp.dot(tc, wh_ref[...]) + bh_ref[...]
        cv_ref[...] = aux_s[...]


def _encoder(t0, p):
    specs = [
        pl.BlockSpec((B, N, D), lambda l, e: (0, 0, 0)),
        pl.BlockSpec((1, 1, D), lambda l, e: (l, 0, 0)),
        pl.BlockSpec((1, 1, D), lambda l, e: (l, 0, 0)),
        pl.BlockSpec((1, D, 3 * D), lambda l, e: (l, 0, 0)),
        pl.BlockSpec((1, 1, 3 * D), lambda l, e: (l, 0, 0)),
        pl.BlockSpec((1, D, D), lambda l, e: (l, 0, 0)),
        pl.BlockSpec((1, 1, D), lambda l, e: (l, 0, 0)),
        pl.BlockSpec((1, 1, D), lambda l, e: (l, 0, 0)),
        pl.BlockSpec((1, 1, D), lambda l, e: (l, 0, 0)),
        pl.BlockSpec((1, D, NE), lambda l, e: (l, 0, 0)),
        pl.BlockSpec((1, 1, D, HD), lambda l, e: (l, e, 0, 0)),
        pl.BlockSpec((1, 1, 1, HD), lambda l, e: (l, e, 0, 0)),
        pl.BlockSpec((1, 1, HD, D), lambda l, e: (l, e, 0, 0)),
        pl.BlockSpec((1, 1, 1, D), lambda l, e: (l, e, 0, 0)),
        pl.BlockSpec((1, D), lambda l, e: (0, 0)),
        pl.BlockSpec((1, D), lambda l, e: (0, 0)),
        pl.BlockSpec((D, 1000), lambda l, e: (0, 0)),
        pl.BlockSpec((1, 1000), lambda l, e: (0, 0)),
    ]
    return pl.pallas_call(
        _encoder_body,
        grid=(L, NE),
        in_specs=specs,
        out_specs=(
            pl.BlockSpec((B, 1000), lambda l, e: (0, 0)),
            pl.BlockSpec((1, 1), lambda l, e: (0, 0)),
        ),
        out_shape=(
            jax.ShapeDtypeStruct((B, 1000), _f32),
            jax.ShapeDtypeStruct((1, 1), _f32),
        ),
        scratch_shapes=[
            pltpu.VMEM((B, NP, D), _f32),
            pltpu.VMEM((BNP, D), _bf16),
            pltpu.VMEM((BNP, D), _bf16),
            pltpu.VMEM((BNP, CAPP), _bf16),
            pltpu.VMEM((BNP,), jnp.int32),
            pltpu.VMEM((BNP, 1), _f32),
            pltpu.VMEM((1, 1), _f32),
        ],
    )(
        t0,
        p['ln1_g'].reshape(L, 1, D), p['ln1_b'].reshape(L, 1, D),
        p['Wqkv'].astype(_bf16), p['bqkv'].reshape(L, 1, 3 * D),
        p['Wproj'].astype(_bf16), p['bproj'].reshape(L, 1, D),
        p['ln2_g'].reshape(L, 1, D), p['ln2_b'].reshape(L, 1, D),
        p['Wg'],
        p['W1'].astype(_bf16), p['b1'].reshape(L, NE, 1, HD),
        p['W2'].astype(_bf16), p['b2'].reshape(L, NE, 1, D),
        p['lnf_g'].reshape(1, D), p['lnf_b'].reshape(1, D),
        p['Whead'], p['bhead'].reshape(1, 1000),
    )


def kernel(x, params):
    p = params
    gh = 224 // 16
    xp = (x.reshape(B, 3, gh, 16, gh, 16)
           .transpose(0, 2, 4, 1, 3, 5)
           .reshape(B * gh * gh, 3 * 16 * 16))
    t0 = _embed(xp, p['Wpatch'], p['bpatch'], p['cls'].reshape(1, D),
                p['pos'].reshape(N, D))
    logits, cv = _encoder(t0, p)
    return logits, cv.reshape(())


# speedup vs baseline: 1.1121x; 1.0773x over previous
import math

import jax
import jax.numpy as jnp
from jax import lax
from jax.experimental import pallas as pl
from jax.experimental.pallas import tpu as pltpu

B = 8
N = 197
D = 192
NH = 3
DH = 64
L = 6
NE = 16
HD = 768
CAP = 197
CAPP = 208
NP = 256
BNP = B * NP

_f32 = jnp.float32
_bf16 = jnp.bfloat16


def _bdot(a, b):
    return lax.dot_general(
        a.astype(_bf16), b.astype(_bf16), (((a.ndim - 1,), (0,)), ((), ())),
        preferred_element_type=_f32)


def _ln(x, g, b):
    m = x.mean(-1, keepdims=True)
    v = ((x - m) ** 2).mean(-1, keepdims=True)
    return (x - m) * lax.rsqrt(v + 1e-6) * g + b


def _softmax(s):
    p = jnp.exp(s)
    return p * (1.0 / jnp.sum(p, axis=-1, keepdims=True))


def _embed_body(xp_ref, wp_ref, bp_ref, cls_ref, pos_ref, out_ref):
    y = jnp.dot(xp_ref[...], wp_ref[...]) + bp_ref[...]
    for b in range(B):
        out_ref[b, 0:1, :] = cls_ref[...] + pos_ref[0:1, :]
        out_ref[b, 1:N, :] = y[b * (N - 1):(b + 1) * (N - 1), :] + pos_ref[1:N, :]


def _embed(xp, wp, bp, cls, pos):
    return pl.pallas_call(
        _embed_body,
        out_shape=jax.ShapeDtypeStruct((B, N, D), _f32),
    )(xp, wp, bp, cls, pos)


def _encoder_body(t0_ref, ln1g_ref, ln1b_ref, wqkv_ref, bqkv_ref, wproj_ref,
                  bproj_ref, ln2g_ref, ln2b_ref, wg_ref, w1_ref, b1_ref,
                  w2_ref, b2_ref, lnfg_ref, lnfb_ref, wh_ref, bh_ref,
                  logits_ref, cv_ref,
                  t_s, u_s, y_s, a_s, idx_s, scl_s, aux_s):
    l = pl.program_id(0)
    e = pl.program_id(1)

    @pl.when(jnp.logical_and(l == 0, e == 0))
    def _zero_pad():
        for b in range(B):
            u_s[pl.ds(b * NP + N, NP - N), :] = jnp.zeros((NP - N, D), _bf16)
            a_s[pl.ds(b * NP + N, NP - N), :] = jnp.zeros((NP - N, CAPP), _bf16)
            idx_s[pl.ds(b * NP + N, NP - N)] = jnp.full((NP - N,), NE, jnp.int32)

    @pl.when(e == 0)
    def _prologue():
        iota_e = lax.broadcasted_iota(jnp.int32, (N, NE), 1).astype(_f32)
        ii = lax.broadcasted_iota(jnp.int32, (N, N), 0)
        jj = lax.broadcasted_iota(jnp.int32, (N, N), 1)
        tril = (jj < ii).astype(_bf16)
        iota_c = lax.broadcasted_iota(jnp.int32, (N, CAPP), 1)
        is_l0 = l == 0

        off = jnp.zeros((NE,), _f32)
        imp = jnp.zeros((NE,), _f32)
        for b in range(B):
            sb = scl_s[pl.ds(b * NP, N), :]
            yrows = y_s[pl.ds(b * NP, N), :].astype(_f32)
            fold = t_s[b, :N, :] + jnp.where(sb > 0.0, yrows * sb, 0.0)
            tb = jnp.where(is_l0, t0_ref[b], fold)
            u1 = _ln(tb, ln1g_ref[0], ln1b_ref[0])
            qkv = _bdot(u1, wqkv_ref[0]) + bqkv_ref[0]
            outs = []
            for h in range(NH):
                q = qkv[:, h * DH:(h + 1) * DH]
                k = qkv[:, D + h * DH:D + (h + 1) * DH]
                v = qkv[:, 2 * D + h * DH:2 * D + (h + 1) * DH]
                s = lax.dot_general(
                    q.astype(_bf16), k.astype(_bf16),
                    (((1,), (1,)), ((), ())),
                    preferred_element_type=_f32) * (1.0 / math.sqrt(DH))
                p = _softmax(s)
                outs.append(_bdot(p, v))
            o = jnp.concatenate(outs, axis=1)
            tm = tb + _bdot(o, wproj_ref[0]) + bproj_ref[0]
            t_s[b, :N, :] = tm
            u2 = _ln(tm, ln2g_ref[0], ln2b_ref[0])
            u_s[pl.ds(b * NP, N), :] = u2.astype(_bf16)

            logits = jnp.dot(u2, wg_ref[0])
            probs = _softmax(logits)
            gate = jnp.max(probs, axis=-1)
            eq = probs == gate[:, None]
            idxf = jnp.min(jnp.where(eq, iota_e, 1e9), axis=-1)
            oh = (iota_e == idxf[:, None]).astype(_f32)
            cnt = _bdot(tril, oh)
            pos = jnp.sum((cnt + off[None, :]) * oh, axis=-1)
            keep = (pos < float(CAP)).astype(_f32)
            posc = jnp.minimum(pos, float(CAPP - 1)).astype(jnp.int32)
            scl_s[pl.ds(b * NP, N), :] = (gate * keep)[:, None]
            a_s[pl.ds(b * NP, N), :] = (posc[:, None] == iota_c).astype(_bf16)
            idx_s[pl.ds(b * NP, N)] = idxf.astype(jnp.int32)
            off = off + jnp.sum(oh, axis=0)
            imp = imp + jnp.sum(probs, axis=0)

        tot = float(B * N)
        aux = float(NE) * jnp.sum((imp / tot) * (off / tot))
        prev = aux_s[0, 0]
        aux_s[...] = jnp.where(is_l0, aux, prev + aux).reshape(1, 1)

    mf = (idx_s[...] == e).astype(_f32)[:, None]
    m = mf.astype(_bf16)
    um = u_s[...] * m
    x = lax.dot_general(a_s[...], um, (((0,), (0,)), ((), ())),
                        preferred_element_type=_f32)
    h = jax.nn.gelu(_bdot(x, w1_ref[0, 0]) + b1_ref[0, 0])
    o = _bdot(h, w2_ref[0, 0]) + b2_ref[0, 0]
    yb = jnp.dot(a_s[...], o.astype(_bf16),
                 preferred_element_type=_f32)
    ym = (yb * mf).astype(_bf16)

    @pl.when(e == 0)
    def _():
        y_s[...] = ym

    @pl.when(e > 0)
    def _():
        y_s[...] += ym

    @pl.when(jnp.logical_and(l == L - 1, e == NE - 1))
    def _head():
        rows = []
        for b in range(B):
            sb = scl_s[pl.ds(b * NP, 1), :]
            ycls = y_s[pl.ds(b * NP, 1), :].astype(_f32)
            tb = t_s[b, 0:1, :] + jnp.where(sb > 0.0, ycls * sb, 0.0)
            rows.append(tb)
        tc = _ln(jnp.concatenate(rows, axis=0), lnfg_ref[0], lnfb_ref[0])
        logits_ref[...] = jnp.dot(tc, wh_ref[...]) + bh_ref[...]
        cv_ref[...] = aux_s[...]


def _encoder(t0, p):
    specs = [
        pl.BlockSpec((B, N, D), lambda l, e: (0, 0, 0)),
        pl.BlockSpec((1, 1, D), lambda l, e: (l, 0, 0)),
        pl.BlockSpec((1, 1, D), lambda l, e: (l, 0, 0)),
        pl.BlockSpec((1, D, 3 * D), lambda l, e: (l, 0, 0)),
        pl.BlockSpec((1, 1, 3 * D), lambda l, e: (l, 0, 0)),
        pl.BlockSpec((1, D, D), lambda l, e: (l, 0, 0)),
        pl.BlockSpec((1, 1, D), lambda l, e: (l, 0, 0)),
        pl.BlockSpec((1, 1, D), lambda l, e: (l, 0, 0)),
        pl.BlockSpec((1, 1, D), lambda l, e: (l, 0, 0)),
        pl.BlockSpec((1, D, NE), lambda l, e: (l, 0, 0)),
        pl.BlockSpec((1, 1, D, HD), lambda l, e: (l, e, 0, 0)),
        pl.BlockSpec((1, 1, 1, HD), lambda l, e: (l, e, 0, 0)),
        pl.BlockSpec((1, 1, HD, D), lambda l, e: (l, e, 0, 0)),
        pl.BlockSpec((1, 1, 1, D), lambda l, e: (l, e, 0, 0)),
        pl.BlockSpec((1, D), lambda l, e: (0, 0)),
        pl.BlockSpec((1, D), lambda l, e: (0, 0)),
        pl.BlockSpec((D, 1000), lambda l, e: (0, 0)),
        pl.BlockSpec((1, 1000), lambda l, e: (0, 0)),
    ]
    return pl.pallas_call(
        _encoder_body,
        grid=(L, NE),
        in_specs=specs,
        out_specs=(
            pl.BlockSpec((B, 1000), lambda l, e: (0, 0)),
            pl.BlockSpec((1, 1), lambda l, e: (0, 0)),
        ),
        out_shape=(
            jax.ShapeDtypeStruct((B, 1000), _f32),
            jax.ShapeDtypeStruct((1, 1), _f32),
        ),
        scratch_shapes=[
            pltpu.VMEM((B, NP, D), _f32),
            pltpu.VMEM((BNP, D), _bf16),
            pltpu.VMEM((BNP, D), _bf16),
            pltpu.VMEM((BNP, CAPP), _bf16),
            pltpu.VMEM((BNP,), jnp.int32),
            pltpu.VMEM((BNP, 1), _f32),
            pltpu.VMEM((1, 1), _f32),
        ],
    )(
        t0,
        p['ln1_g'].reshape(L, 1, D), p['ln1_b'].reshape(L, 1, D),
        p['Wqkv'], p['bqkv'].reshape(L, 1, 3 * D),
        p['Wproj'], p['bproj'].reshape(L, 1, D),
        p['ln2_g'].reshape(L, 1, D), p['ln2_b'].reshape(L, 1, D),
        p['Wg'],
        p['W1'], p['b1'].reshape(L, NE, 1, HD),
        p['W2'], p['b2'].reshape(L, NE, 1, D),
        p['lnf_g'].reshape(1, D), p['lnf_b'].reshape(1, D),
        p['Whead'], p['bhead'].reshape(1, 1000),
    )


def kernel(x, params):
    p = params
    gh = 224 // 16
    xp = (x.reshape(B, 3, gh, 16, gh, 16)
           .transpose(0, 2, 4, 1, 3, 5)
           .reshape(B * gh * gh, 3 * 16 * 16))
    t0 = _embed(xp, p['Wpatch'], p['bpatch'], p['cls'].reshape(1, D),
                p['pos'].reshape(N, D))
    logits, cv = _encoder(t0, p)
    return logits, cv.reshape(())


# speedup vs baseline: 1.1777x; 1.0590x over previous
import math

import jax
import jax.numpy as jnp
from jax import lax
from jax.experimental import pallas as pl
from jax.experimental.pallas import tpu as pltpu

B = 8
N = 197
D = 192
NH = 3
DH = 64
L = 6
NE = 16
HD = 768
CAP = 197
CAPP = 208
NP = 256
BNP = B * NP

_f32 = jnp.float32
_bf16 = jnp.bfloat16


def _bdot(a, b):
    return lax.dot_general(
        a.astype(_bf16), b.astype(_bf16), (((a.ndim - 1,), (0,)), ((), ())),
        preferred_element_type=_f32)


def _ln(x, g, b):
    m = x.mean(-1, keepdims=True)
    v = ((x - m) ** 2).mean(-1, keepdims=True)
    return (x - m) * lax.rsqrt(v + 1e-6) * g + b


def _softmax(s):
    p = jnp.exp(s)
    return p * (1.0 / jnp.sum(p, axis=-1, keepdims=True))


def _embed_body(xp_ref, wp_ref, bp_ref, cls_ref, pos_ref, out_ref):
    y = jnp.dot(xp_ref[...], wp_ref[...]) + bp_ref[...]
    for b in range(B):
        out_ref[b, 0:1, :] = cls_ref[...] + pos_ref[0:1, :]
        out_ref[b, 1:N, :] = y[b * (N - 1):(b + 1) * (N - 1), :] + pos_ref[1:N, :]


def _embed(xp, wp, bp, cls, pos):
    return pl.pallas_call(
        _embed_body,
        out_shape=jax.ShapeDtypeStruct((B, N, D), _f32),
    )(xp, wp, bp, cls, pos)


def _encoder_body(t0_ref, ln1g_ref, ln1b_ref, wqkv_ref, bqkv_ref, wproj_ref,
                  bproj_ref, ln2g_ref, ln2b_ref, wg_ref, w1_ref, b1_ref,
                  w2_ref, b2_ref, lnfg_ref, lnfb_ref, wh_ref, bh_ref,
                  logits_ref, cv_ref,
                  t_s, u_s, y_s, a_s, idx_s, scl_s, aux_s):
    l = pl.program_id(0)
    e2 = pl.program_id(1)

    @pl.when(jnp.logical_and(l == 0, e2 == 0))
    def _zero_pad():
        for b in range(B):
            u_s[pl.ds(b * NP + N, NP - N), :] = jnp.zeros((NP - N, D), _bf16)
            a_s[pl.ds(b * NP + N, NP - N), :] = jnp.zeros((NP - N, CAPP), _bf16)
            idx_s[pl.ds(b * NP + N, NP - N)] = jnp.full((NP - N,), NE, jnp.int32)

    @pl.when(e2 == 0)
    def _prologue():
        iota_e = lax.broadcasted_iota(jnp.int32, (N, NE), 1).astype(_f32)
        ii = lax.broadcasted_iota(jnp.int32, (N, N), 0)
        jj = lax.broadcasted_iota(jnp.int32, (N, N), 1)
        tril = (jj < ii).astype(_bf16)
        iota_c = lax.broadcasted_iota(jnp.int32, (N, CAPP), 1)
        is_l0 = l == 0

        off = jnp.zeros((NE,), _f32)
        imp = jnp.zeros((NE,), _f32)
        for b in range(B):
            sb = scl_s[pl.ds(b * NP, N), :]
            yrows = y_s[pl.ds(b * NP, N), :].astype(_f32)
            fold = t_s[b, :N, :] + jnp.where(sb > 0.0, yrows * sb, 0.0)
            tb = jnp.where(is_l0, t0_ref[b], fold)
            u1 = _ln(tb, ln1g_ref[0], ln1b_ref[0])
            qkv = _bdot(u1, wqkv_ref[0]) + bqkv_ref[0]
            outs = []
            for h in range(NH):
                q = qkv[:, h * DH:(h + 1) * DH]
                k = qkv[:, D + h * DH:D + (h + 1) * DH]
                v = qkv[:, 2 * D + h * DH:2 * D + (h + 1) * DH]
                s = lax.dot_general(
                    q.astype(_bf16), k.astype(_bf16),
                    (((1,), (1,)), ((), ())),
                    preferred_element_type=_f32) * (1.0 / math.sqrt(DH))
                p = _softmax(s)
                outs.append(_bdot(p, v))
            o = jnp.concatenate(outs, axis=1)
            tm = tb + _bdot(o, wproj_ref[0]) + bproj_ref[0]
            t_s[b, :N, :] = tm
            u2 = _ln(tm, ln2g_ref[0], ln2b_ref[0])
            u_s[pl.ds(b * NP, N), :] = u2.astype(_bf16)

            logits = jnp.dot(u2, wg_ref[0])
            probs = _softmax(logits)
            gate = jnp.max(probs, axis=-1)
            eq = probs == gate[:, None]
            idxf = jnp.min(jnp.where(eq, iota_e, 1e9), axis=-1)
            oh = (iota_e == idxf[:, None]).astype(_f32)
            cnt = _bdot(tril, oh)
            pos = jnp.sum((cnt + off[None, :]) * oh, axis=-1)
            keep = (pos < float(CAP)).astype(_f32)
            posc = jnp.minimum(pos, float(CAPP - 1)).astype(jnp.int32)
            scl_s[pl.ds(b * NP, N), :] = (gate * keep)[:, None]
            a_s[pl.ds(b * NP, N), :] = (posc[:, None] == iota_c).astype(_bf16)
            idx_s[pl.ds(b * NP, N)] = idxf.astype(jnp.int32)
            off = off + jnp.sum(oh, axis=0)
            imp = imp + jnp.sum(probs, axis=0)

        tot = float(B * N)
        aux = float(NE) * jnp.sum((imp / tot) * (off / tot))
        prev = aux_s[0, 0]
        aux_s[...] = jnp.where(is_l0, aux, prev + aux).reshape(1, 1)

    yms = []
    for ee in range(2):
        e = e2 * 2 + ee
        mf = (idx_s[...] == e).astype(_f32)[:, None]
        m = mf.astype(_bf16)
        um = u_s[...] * m
        x = lax.dot_general(a_s[...], um, (((0,), (0,)), ((), ())),
                            preferred_element_type=_f32)
        h = jax.nn.gelu(_bdot(x, w1_ref[0, ee]) + b1_ref[0, ee])
        o = _bdot(h, w2_ref[0, ee]) + b2_ref[0, ee]
        yb = jnp.dot(a_s[...], o.astype(_bf16),
                     preferred_element_type=_f32)
        yms.append(yb * mf)
    ym = (yms[0] + yms[1]).astype(_bf16)

    @pl.when(e2 == 0)
    def _():
        y_s[...] = ym

    @pl.when(e2 > 0)
    def _():
        y_s[...] += ym

    @pl.when(jnp.logical_and(l == L - 1, e2 == NE // 2 - 1))
    def _head():
        rows = []
        for b in range(B):
            sb = scl_s[pl.ds(b * NP, 1), :]
            ycls = y_s[pl.ds(b * NP, 1), :].astype(_f32)
            tb = t_s[b, 0:1, :] + jnp.where(sb > 0.0, ycls * sb, 0.0)
            rows.append(tb)
        tc = _ln(jnp.concatenate(rows, axis=0), lnfg_ref[0], lnfb_ref[0])
        logits_ref[...] = jnp.dot(tc, wh_ref[...]) + bh_ref[...]
        cv_ref[...] = aux_s[...]


def _encoder(t0, p):
    specs = [
        pl.BlockSpec((B, N, D), lambda l, e: (0, 0, 0)),
        pl.BlockSpec((1, 1, D), lambda l, e: (l, 0, 0)),
        pl.BlockSpec((1, 1, D), lambda l, e: (l, 0, 0)),
        pl.BlockSpec((1, D, 3 * D), lambda l, e: (l, 0, 0)),
        pl.BlockSpec((1, 1, 3 * D), lambda l, e: (l, 0, 0)),
        pl.BlockSpec((1, D, D), lambda l, e: (l, 0, 0)),
        pl.BlockSpec((1, 1, D), lambda l, e: (l, 0, 0)),
        pl.BlockSpec((1, 1, D), lambda l, e: (l, 0, 0)),
        pl.BlockSpec((1, 1, D), lambda l, e: (l, 0, 0)),
        pl.BlockSpec((1, D, NE), lambda l, e: (l, 0, 0)),
        pl.BlockSpec((1, 2, D, HD), lambda l, e: (l, e, 0, 0)),
        pl.BlockSpec((1, 2, 1, HD), lambda l, e: (l, e, 0, 0)),
        pl.BlockSpec((1, 2, HD, D), lambda l, e: (l, e, 0, 0)),
        pl.BlockSpec((1, 2, 1, D), lambda l, e: (l, e, 0, 0)),
        pl.BlockSpec((1, D), lambda l, e: (0, 0)),
        pl.BlockSpec((1, D), lambda l, e: (0, 0)),
        pl.BlockSpec((D, 1000), lambda l, e: (0, 0)),
        pl.BlockSpec((1, 1000), lambda l, e: (0, 0)),
    ]
    return pl.pallas_call(
        _encoder_body,
        grid=(L, NE // 2),
        in_specs=specs,
        out_specs=(
            pl.BlockSpec((B, 1000), lambda l, e: (0, 0)),
            pl.BlockSpec((1, 1), lambda l, e: (0, 0)),
        ),
        out_shape=(
            jax.ShapeDtypeStruct((B, 1000), _f32),
            jax.ShapeDtypeStruct((1, 1), _f32),
        ),
        scratch_shapes=[
            pltpu.VMEM((B, NP, D), _f32),
            pltpu.VMEM((BNP, D), _bf16),
            pltpu.VMEM((BNP, D), _bf16),
            pltpu.VMEM((BNP, CAPP), _bf16),
            pltpu.VMEM((BNP,), jnp.int32),
            pltpu.VMEM((BNP, 1), _f32),
            pltpu.VMEM((1, 1), _f32),
        ],
    )(
        t0,
        p['ln1_g'].reshape(L, 1, D), p['ln1_b'].reshape(L, 1, D),
        p['Wqkv'], p['bqkv'].reshape(L, 1, 3 * D),
        p['Wproj'], p['bproj'].reshape(L, 1, D),
        p['ln2_g'].reshape(L, 1, D), p['ln2_b'].reshape(L, 1, D),
        p['Wg'],
        p['W1'], p['b1'].reshape(L, NE, 1, HD),
        p['W2'], p['b2'].reshape(L, NE, 1, D),
        p['lnf_g'].reshape(1, D), p['lnf_b'].reshape(1, D),
        p['Whead'], p['bhead'].reshape(1, 1000),
    )


def kernel(x, params):
    p = params
    gh = 224 // 16
    xp = (x.reshape(B, 3, gh, 16, gh, 16)
           .transpose(0, 2, 4, 1, 3, 5)
           .reshape(B * gh * gh, 3 * 16 * 16))
    t0 = _embed(xp, p['Wpatch'], p['bpatch'], p['cls'].reshape(1, D),
                p['pos'].reshape(N, D))
    logits, cv = _encoder(t0, p)
    return logits, cv.reshape(())


# speedup vs baseline: 1.2802x; 1.0870x over previous
import math

import jax
import jax.numpy as jnp
from jax import lax
from jax.experimental import pallas as pl
from jax.experimental.pallas import tpu as pltpu

B = 8
N = 197
D = 192
NH = 3
DH = 64
L = 6
NE = 16
HD = 768
CAP = 197
CAPP = 208
NP = 256
BNP = B * NP

_f32 = jnp.float32
_bf16 = jnp.bfloat16


def _bdot(a, b):
    return lax.dot_general(
        a.astype(_bf16), b.astype(_bf16), (((a.ndim - 1,), (0,)), ((), ())),
        preferred_element_type=_f32)


def _ln(x, g, b):
    m = x.mean(-1, keepdims=True)
    v = ((x - m) ** 2).mean(-1, keepdims=True)
    return (x - m) * lax.rsqrt(v + 1e-6) * g + b


def _softmax(s):
    p = jnp.exp(s)
    return p * (1.0 / jnp.sum(p, axis=-1, keepdims=True))


def _embed_body(xp_ref, wp_ref, bp_ref, cls_ref, pos_ref, out_ref):
    y = jnp.dot(xp_ref[...], wp_ref[...]) + bp_ref[...]
    for b in range(B):
        out_ref[b, 0:1, :] = cls_ref[...] + pos_ref[0:1, :]
        out_ref[b, 1:N, :] = y[b * (N - 1):(b + 1) * (N - 1), :] + pos_ref[1:N, :]


def _embed(xp, wp, bp, cls, pos):
    return pl.pallas_call(
        _embed_body,
        out_shape=jax.ShapeDtypeStruct((B, N, D), _f32),
    )(xp, wp, bp, cls, pos)


def _encoder_body(t0_ref, ln1g_ref, ln1b_ref, wqkv_ref, bqkv_ref, wproj_ref,
                  bproj_ref, ln2g_ref, ln2b_ref, wg_ref, w1_ref, b1_ref,
                  w2_ref, b2_ref, lnfg_ref, lnfb_ref, wh_ref, bh_ref,
                  logits_ref, cv_ref,
                  t_s, u_s, y_s, a_s, idx_s, scl_s, aux_s):
    l = pl.program_id(0)
    e2 = pl.program_id(1)

    @pl.when(jnp.logical_and(l == 0, e2 == 0))
    def _zero_pad():
        for b in range(B):
            u_s[pl.ds(b * NP + N, NP - N), :] = jnp.zeros((NP - N, D), _bf16)
            a_s[pl.ds(b * NP + N, NP - N), :] = jnp.zeros((NP - N, CAPP), _bf16)
            idx_s[pl.ds(b * NP + N, NP - N)] = jnp.full((NP - N,), NE, jnp.int32)

    @pl.when(e2 == 0)
    def _prologue():
        iota_e = lax.broadcasted_iota(jnp.int32, (N, NE), 1).astype(_f32)
        ii = lax.broadcasted_iota(jnp.int32, (N, N), 0)
        jj = lax.broadcasted_iota(jnp.int32, (N, N), 1)
        tril = (jj < ii).astype(_bf16)
        iota_c = lax.broadcasted_iota(jnp.int32, (N, CAPP), 1)
        is_l0 = l == 0

        off = jnp.zeros((NE,), _f32)
        imp = jnp.zeros((NE,), _f32)
        for b in range(B):
            sb = scl_s[pl.ds(b * NP, N), :]
            yrows = y_s[pl.ds(b * NP, N), :].astype(_f32)
            fold = t_s[b, :N, :] + jnp.where(sb > 0.0, yrows * sb, 0.0)
            tb = jnp.where(is_l0, t0_ref[b], fold)
            u1 = _ln(tb, ln1g_ref[0], ln1b_ref[0])
            qkv = _bdot(u1, wqkv_ref[0]) + bqkv_ref[0]
            outs = []
            for h in range(NH):
                q = qkv[:, h * DH:(h + 1) * DH]
                k = qkv[:, D + h * DH:D + (h + 1) * DH]
                v = qkv[:, 2 * D + h * DH:2 * D + (h + 1) * DH]
                s = lax.dot_general(
                    q.astype(_bf16), k.astype(_bf16),
                    (((1,), (1,)), ((), ())),
                    preferred_element_type=_f32) * (1.0 / math.sqrt(DH))
                p = _softmax(s)
                outs.append(_bdot(p, v))
            o = jnp.concatenate(outs, axis=1)
            tm = tb + _bdot(o, wproj_ref[0]) + bproj_ref[0]
            t_s[b, :N, :] = tm
            u2 = _ln(tm, ln2g_ref[0], ln2b_ref[0])
            u_s[pl.ds(b * NP, N), :] = u2.astype(_bf16)

            logits = jnp.dot(u2, wg_ref[0])
            probs = _softmax(logits)
            gate = jnp.max(probs, axis=-1)
            eq = probs == gate[:, None]
            idxf = jnp.min(jnp.where(eq, iota_e, 1e9), axis=-1)
            oh = (iota_e == idxf[:, None]).astype(_f32)
            cnt = _bdot(tril, oh)
            pos = jnp.sum((cnt + off[None, :]) * oh, axis=-1)
            keep = (pos < float(CAP)).astype(_f32)
            posc = jnp.minimum(pos, float(CAPP - 1)).astype(jnp.int32)
            scl_s[pl.ds(b * NP, N), :] = (gate * keep)[:, None]
            a_s[pl.ds(b * NP, N), :] = (posc[:, None] == iota_c).astype(_bf16)
            idx_s[pl.ds(b * NP, N)] = idxf.astype(jnp.int32)
            off = off + jnp.sum(oh, axis=0)
            imp = imp + jnp.sum(probs, axis=0)

        tot = float(B * N)
        aux = float(NE) * jnp.sum((imp / tot) * (off / tot))
        prev = aux_s[0, 0]
        aux_s[...] = jnp.where(is_l0, aux, prev + aux).reshape(1, 1)

    yms = []
    for ee in range(4):
        e = e2 * 4 + ee
        mf = (idx_s[...] == e).astype(_f32)[:, None]
        m = mf.astype(_bf16)
        um = u_s[...] * m
        x = lax.dot_general(a_s[...], um, (((0,), (0,)), ((), ())),
                            preferred_element_type=_f32)
        h = jax.nn.gelu(_bdot(x, w1_ref[0, ee]) + b1_ref[0, ee])
        o = _bdot(h, w2_ref[0, ee]) + b2_ref[0, ee]
        yb = jnp.dot(a_s[...], o.astype(_bf16),
                     preferred_element_type=_f32)
        yms.append(yb * mf)
    ym = ((yms[0] + yms[1]) + (yms[2] + yms[3])).astype(_bf16)

    @pl.when(e2 == 0)
    def _():
        y_s[...] = ym

    @pl.when(e2 > 0)
    def _():
        y_s[...] += ym

    @pl.when(jnp.logical_and(l == L - 1, e2 == NE // 4 - 1))
    def _head():
        rows = []
        for b in range(B):
            sb = scl_s[pl.ds(b * NP, 1), :]
            ycls = y_s[pl.ds(b * NP, 1), :].astype(_f32)
            tb = t_s[b, 0:1, :] + jnp.where(sb > 0.0, ycls * sb, 0.0)
            rows.append(tb)
        tc = _ln(jnp.concatenate(rows, axis=0), lnfg_ref[0], lnfb_ref[0])
        logits_ref[...] = jnp.dot(tc, wh_ref[...]) + bh_ref[...]
        cv_ref[...] = aux_s[...]


def _encoder(t0, p):
    specs = [
        pl.BlockSpec((B, N, D), lambda l, e: (0, 0, 0)),
        pl.BlockSpec((1, 1, D), lambda l, e: (l, 0, 0)),
        pl.BlockSpec((1, 1, D), lambda l, e: (l, 0, 0)),
        pl.BlockSpec((1, D, 3 * D), lambda l, e: (l, 0, 0)),
        pl.BlockSpec((1, 1, 3 * D), lambda l, e: (l, 0, 0)),
        pl.BlockSpec((1, D, D), lambda l, e: (l, 0, 0)),
        pl.BlockSpec((1, 1, D), lambda l, e: (l, 0, 0)),
        pl.BlockSpec((1, 1, D), lambda l, e: (l, 0, 0)),
        pl.BlockSpec((1, 1, D), lambda l, e: (l, 0, 0)),
        pl.BlockSpec((1, D, NE), lambda l, e: (l, 0, 0)),
        pl.BlockSpec((1, 4, D, HD), lambda l, e: (l, e, 0, 0)),
        pl.BlockSpec((1, 4, 1, HD), lambda l, e: (l, e, 0, 0)),
        pl.BlockSpec((1, 4, HD, D), lambda l, e: (l, e, 0, 0)),
        pl.BlockSpec((1, 4, 1, D), lambda l, e: (l, e, 0, 0)),
        pl.BlockSpec((1, D), lambda l, e: (0, 0)),
        pl.BlockSpec((1, D), lambda l, e: (0, 0)),
        pl.BlockSpec((D, 1000), lambda l, e: (0, 0)),
        pl.BlockSpec((1, 1000), lambda l, e: (0, 0)),
    ]
    return pl.pallas_call(
        _encoder_body,
        grid=(L, NE // 4),
        in_specs=specs,
        out_specs=(
            pl.BlockSpec((B, 1000), lambda l, e: (0, 0)),
            pl.BlockSpec((1, 1), lambda l, e: (0, 0)),
        ),
        out_shape=(
            jax.ShapeDtypeStruct((B, 1000), _f32),
            jax.ShapeDtypeStruct((1, 1), _f32),
        ),
        scratch_shapes=[
            pltpu.VMEM((B, NP, D), _f32),
            pltpu.VMEM((BNP, D), _bf16),
            pltpu.VMEM((BNP, D), _bf16),
            pltpu.VMEM((BNP, CAPP), _bf16),
            pltpu.VMEM((BNP,), jnp.int32),
            pltpu.VMEM((BNP, 1), _f32),
            pltpu.VMEM((1, 1), _f32),
        ],
    )(
        t0,
        p['ln1_g'].reshape(L, 1, D), p['ln1_b'].reshape(L, 1, D),
        p['Wqkv'], p['bqkv'].reshape(L, 1, 3 * D),
        p['Wproj'], p['bproj'].reshape(L, 1, D),
        p['ln2_g'].reshape(L, 1, D), p['ln2_b'].reshape(L, 1, D),
        p['Wg'],
        p['W1'], p['b1'].reshape(L, NE, 1, HD),
        p['W2'], p['b2'].reshape(L, NE, 1, D),
        p['lnf_g'].reshape(1, D), p['lnf_b'].reshape(1, D),
        p['Whead'], p['bhead'].reshape(1, 1000),
    )


def kernel(x, params):
    p = params
    gh = 224 // 16
    xp = (x.reshape(B, 3, gh, 16, gh, 16)
           .transpose(0, 2, 4, 1, 3, 5)
           .reshape(B * gh * gh, 3 * 16 * 16))
    t0 = _embed(xp, p['Wpatch'], p['bpatch'], p['cls'].reshape(1, D),
                p['pos'].reshape(N, D))
    logits, cv = _encoder(t0, p)
    return logits, cv.reshape(())


# speedup vs baseline: 1.3382x; 1.0453x over previous
import math

import jax
import jax.numpy as jnp
from jax import lax
from jax.experimental import pallas as pl
from jax.experimental.pallas import tpu as pltpu

B = 8
N = 197
D = 192
NH = 3
DH = 64
L = 6
NE = 16
HD = 768
CAP = 197
CAPP = 208
NP = 256
BNP = B * NP

_f32 = jnp.float32
_bf16 = jnp.bfloat16


def _bdot(a, b):
    return lax.dot_general(
        a.astype(_bf16), b.astype(_bf16), (((a.ndim - 1,), (0,)), ((), ())),
        preferred_element_type=_f32)


def _ln(x, g, b):
    m = x.mean(-1, keepdims=True)
    v = ((x - m) ** 2).mean(-1, keepdims=True)
    return (x - m) * lax.rsqrt(v + 1e-6) * g + b


def _softmax(s):
    p = jnp.exp(s)
    return p * (1.0 / jnp.sum(p, axis=-1, keepdims=True))


def _embed_body(xp_ref, wp_ref, bp_ref, cls_ref, pos_ref, out_ref):
    y = jnp.dot(xp_ref[...], wp_ref[...]) + bp_ref[...]
    for b in range(B):
        out_ref[b, 0:1, :] = cls_ref[...] + pos_ref[0:1, :]
        out_ref[b, 1:N, :] = y[b * (N - 1):(b + 1) * (N - 1), :] + pos_ref[1:N, :]


def _embed(xp, wp, bp, cls, pos):
    return pl.pallas_call(
        _embed_body,
        out_shape=jax.ShapeDtypeStruct((B, N, D), _f32),
    )(xp, wp, bp, cls, pos)


def _encoder_body(t0_ref, ln1g_ref, ln1b_ref, wqkv_ref, bqkv_ref, wproj_ref,
                  bproj_ref, ln2g_ref, ln2b_ref, wg_ref, w1_ref, b1_ref,
                  w2_ref, b2_ref, lnfg_ref, lnfb_ref, wh_ref, bh_ref,
                  logits_ref, cv_ref,
                  t_s, u_s, y_s, a_s, idx_s, scl_s, aux_s):
    l = pl.program_id(0)
    e2 = pl.program_id(1)

    @pl.when(jnp.logical_and(l == 0, e2 == 0))
    def _zero_pad():
        for b in range(B):
            u_s[pl.ds(b * NP + N, NP - N), :] = jnp.zeros((NP - N, D), _bf16)
            a_s[pl.ds(b * NP + N, NP - N), :] = jnp.zeros((NP - N, CAPP), _bf16)
            idx_s[pl.ds(b * NP + N, NP - N)] = jnp.full((NP - N,), NE, jnp.int32)

    @pl.when(e2 == 0)
    def _prologue():
        iota_e = lax.broadcasted_iota(jnp.int32, (N, NE), 1).astype(_f32)
        ii = lax.broadcasted_iota(jnp.int32, (N, N), 0)
        jj = lax.broadcasted_iota(jnp.int32, (N, N), 1)
        tril = (jj < ii).astype(_bf16)
        iota_c = lax.broadcasted_iota(jnp.int32, (N, CAPP), 1)
        is_l0 = l == 0

        off = jnp.zeros((NE,), _f32)
        imp = jnp.zeros((NE,), _f32)
        for b in range(B):
            sb = scl_s[pl.ds(b * NP, N), :]
            yrows = y_s[pl.ds(b * NP, N), :].astype(_f32)
            fold = t_s[b, :N, :] + jnp.where(sb > 0.0, yrows * sb, 0.0)
            tb = jnp.where(is_l0, t0_ref[b], fold)
            u1 = _ln(tb, ln1g_ref[0], ln1b_ref[0])
            qkv = _bdot(u1, wqkv_ref[0]) + bqkv_ref[0]
            outs = []
            for h in range(NH):
                q = qkv[:, h * DH:(h + 1) * DH]
                k = qkv[:, D + h * DH:D + (h + 1) * DH]
                v = qkv[:, 2 * D + h * DH:2 * D + (h + 1) * DH]
                s = lax.dot_general(
                    q.astype(_bf16), k.astype(_bf16),
                    (((1,), (1,)), ((), ())),
                    preferred_element_type=_f32) * (1.0 / math.sqrt(DH))
                p = _softmax(s)
                outs.append(_bdot(p, v))
            o = jnp.concatenate(outs, axis=1)
            tm = tb + _bdot(o, wproj_ref[0]) + bproj_ref[0]
            t_s[b, :N, :] = tm
            u2 = _ln(tm, ln2g_ref[0], ln2b_ref[0])
            u_s[pl.ds(b * NP, N), :] = u2.astype(_bf16)

            logits = jnp.dot(u2, wg_ref[0])
            probs = _softmax(logits)
            gate = jnp.max(probs, axis=-1)
            eq = probs == gate[:, None]
            idxf = jnp.min(jnp.where(eq, iota_e, 1e9), axis=-1)
            oh = (iota_e == idxf[:, None]).astype(_f32)
            cnt = _bdot(tril, oh)
            pos = jnp.sum((cnt + off[None, :]) * oh, axis=-1)
            keep = (pos < float(CAP)).astype(_f32)
            posc = jnp.minimum(pos, float(CAPP - 1)).astype(jnp.int32)
            scl_s[pl.ds(b * NP, N), :] = (gate * keep)[:, None]
            a_s[pl.ds(b * NP, N), :] = (posc[:, None] == iota_c).astype(_bf16)
            idx_s[pl.ds(b * NP, N)] = idxf.astype(jnp.int32)
            off = off + jnp.sum(oh, axis=0)
            imp = imp + jnp.sum(probs, axis=0)

        tot = float(B * N)
        aux = float(NE) * jnp.sum((imp / tot) * (off / tot))
        prev = aux_s[0, 0]
        aux_s[...] = jnp.where(is_l0, aux, prev + aux).reshape(1, 1)

    yms = []
    for ee in range(8):
        e = e2 * 8 + ee
        mf = (idx_s[...] == e).astype(_f32)[:, None]
        m = mf.astype(_bf16)
        um = u_s[...] * m
        x = lax.dot_general(a_s[...], um, (((0,), (0,)), ((), ())),
                            preferred_element_type=_f32)
        h = jax.nn.gelu(_bdot(x, w1_ref[0, ee]) + b1_ref[0, ee])
        o = _bdot(h, w2_ref[0, ee]) + b2_ref[0, ee]
        yb = jnp.dot(a_s[...], o.astype(_bf16),
                     preferred_element_type=_f32)
        yms.append(yb * mf)
    ym = (((yms[0] + yms[1]) + (yms[2] + yms[3])) + ((yms[4] + yms[5]) + (yms[6] + yms[7]))).astype(_bf16)

    @pl.when(e2 == 0)
    def _():
        y_s[...] = ym

    @pl.when(e2 > 0)
    def _():
        y_s[...] += ym

    @pl.when(jnp.logical_and(l == L - 1, e2 == NE // 8 - 1))
    def _head():
        rows = []
        for b in range(B):
            sb = scl_s[pl.ds(b * NP, 1), :]
            ycls = y_s[pl.ds(b * NP, 1), :].astype(_f32)
            tb = t_s[b, 0:1, :] + jnp.where(sb > 0.0, ycls * sb, 0.0)
            rows.append(tb)
        tc = _ln(jnp.concatenate(rows, axis=0), lnfg_ref[0], lnfb_ref[0])
        logits_ref[...] = jnp.dot(tc, wh_ref[...]) + bh_ref[...]
        cv_ref[...] = aux_s[...]


def _encoder(t0, p):
    specs = [
        pl.BlockSpec((B, N, D), lambda l, e: (0, 0, 0)),
        pl.BlockSpec((1, 1, D), lambda l, e: (l, 0, 0)),
        pl.BlockSpec((1, 1, D), lambda l, e: (l, 0, 0)),
        pl.BlockSpec((1, D, 3 * D), lambda l, e: (l, 0, 0)),
        pl.BlockSpec((1, 1, 3 * D), lambda l, e: (l, 0, 0)),
        pl.BlockSpec((1, D, D), lambda l, e: (l, 0, 0)),
        pl.BlockSpec((1, 1, D), lambda l, e: (l, 0, 0)),
        pl.BlockSpec((1, 1, D), lambda l, e: (l, 0, 0)),
        pl.BlockSpec((1, 1, D), lambda l, e: (l, 0, 0)),
        pl.BlockSpec((1, D, NE), lambda l, e: (l, 0, 0)),
        pl.BlockSpec((1, 8, D, HD), lambda l, e: (l, e, 0, 0)),
        pl.BlockSpec((1, 8, 1, HD), lambda l, e: (l, e, 0, 0)),
        pl.BlockSpec((1, 8, HD, D), lambda l, e: (l, e, 0, 0)),
        pl.BlockSpec((1, 8, 1, D), lambda l, e: (l, e, 0, 0)),
        pl.BlockSpec((1, D), lambda l, e: (0, 0)),
        pl.BlockSpec((1, D), lambda l, e: (0, 0)),
        pl.BlockSpec((D, 1000), lambda l, e: (0, 0)),
        pl.BlockSpec((1, 1000), lambda l, e: (0, 0)),
    ]
    return pl.pallas_call(
        _encoder_body,
        grid=(L, NE // 8),
        in_specs=specs,
        out_specs=(
            pl.BlockSpec((B, 1000), lambda l, e: (0, 0)),
            pl.BlockSpec((1, 1), lambda l, e: (0, 0)),
        ),
        out_shape=(
            jax.ShapeDtypeStruct((B, 1000), _f32),
            jax.ShapeDtypeStruct((1, 1), _f32),
        ),
        scratch_shapes=[
            pltpu.VMEM((B, NP, D), _f32),
            pltpu.VMEM((BNP, D), _bf16),
            pltpu.VMEM((BNP, D), _bf16),
            pltpu.VMEM((BNP, CAPP), _bf16),
            pltpu.VMEM((BNP,), jnp.int32),
            pltpu.VMEM((BNP, 1), _f32),
            pltpu.VMEM((1, 1), _f32),
        ],
    )(
        t0,
        p['ln1_g'].reshape(L, 1, D), p['ln1_b'].reshape(L, 1, D),
        p['Wqkv'], p['bqkv'].reshape(L, 1, 3 * D),
        p['Wproj'], p['bproj'].reshape(L, 1, D),
        p['ln2_g'].reshape(L, 1, D), p['ln2_b'].reshape(L, 1, D),
        p['Wg'],
        p['W1'], p['b1'].reshape(L, NE, 1, HD),
        p['W2'], p['b2'].reshape(L, NE, 1, D),
        p['lnf_g'].reshape(1, D), p['lnf_b'].reshape(1, D),
        p['Whead'], p['bhead'].reshape(1, 1000),
    )


def kernel(x, params):
    p = params
    gh = 224 // 16
    xp = (x.reshape(B, 3, gh, 16, gh, 16)
           .transpose(0, 2, 4, 1, 3, 5)
           .reshape(B * gh * gh, 3 * 16 * 16))
    t0 = _embed(xp, p['Wpatch'], p['bpatch'], p['cls'].reshape(1, D),
                p['pos'].reshape(N, D))
    logits, cv = _encoder(t0, p)
    return logits, cv.reshape(())
